# baseline (speedup 1.0000x reference)
import functools

import jax
import jax.numpy as jnp
from jax import lax
from jax.experimental import pallas as pl
from jax.experimental.pallas import tpu as pltpu
from jax.experimental.pallas import tpu_sc as plsc

NC = 2
NS = 16
NW = NC * NS
B = 16384
D = 16
BPW = B // NW
CHUNK = 128
NCHUNK = BPW // CHUNK


def _gmf_body(u_idx_hbm, i_idx_hbm, ut_hbm, it_hbm, out_hbm,
              idx_u_v, idx_i_v, rows_u_v, rows_i_v, sem_u, sem_i):
    wid = lax.axis_index("s") * NC + lax.axis_index("c")
    base = wid * BPW

    pltpu.sync_copy(u_idx_hbm.at[wid], idx_u_v)
    pltpu.sync_copy(i_idx_hbm.at[wid], idx_i_v)

    copies = []
    for j in range(NCHUNK):
        dst = pl.ds(j * CHUNK, CHUNK)
        copies.append(
            pltpu.async_copy(ut_hbm.at[idx_u_v.at[j]], rows_u_v.at[dst], sem_u))
        copies.append(
            pltpu.async_copy(it_hbm.at[idx_i_v.at[j]], rows_i_v.at[dst], sem_i))
    for c in copies:
        c.wait()

    @pl.loop(0, BPW)
    def _(r):
        rows_u_v[r] = rows_u_v[r] * rows_i_v[r]

    pltpu.sync_copy(rows_u_v, out_hbm.at[pl.ds(base, BPW)])


def kernel(inputs, user_table, item_table):
    u_idx = inputs[:, 0].astype(jnp.int32).reshape(NW, NCHUNK, CHUNK)
    i_idx = inputs[:, 1].astype(jnp.int32).reshape(NW, NCHUNK, CHUNK)

    run = pl.kernel(
        _gmf_body,
        out_type=jax.ShapeDtypeStruct((B, D), jnp.float32),
        mesh=plsc.VectorSubcoreMesh(core_axis_name="c", subcore_axis_name="s"),
        compiler_params=pltpu.CompilerParams(use_tc_tiling_on_sc=False),
        scratch_types=[
            pltpu.VMEM((NCHUNK, CHUNK), jnp.int32),
            pltpu.VMEM((NCHUNK, CHUNK), jnp.int32),
            pltpu.VMEM((BPW, D), jnp.float32),
            pltpu.VMEM((BPW, D), jnp.float32),
            pltpu.SemaphoreType.DMA,
            pltpu.SemaphoreType.DMA,
        ],
    )
    return run(u_idx, i_idx, user_table, item_table)
